# Initial kernel scaffold; baseline (speedup 1.0000x reference)
import jax
import jax.numpy as jnp
from jax.experimental import pallas as pl


def kernel(x, router_w, w1, b1, w2, b2, ln_g, ln_b):
    raise NotImplementedError("write your pallas kernel here")


# speedup vs baseline: 4.8462x; 4.8462x over previous
import functools

import jax
import jax.numpy as jnp
from jax.experimental import pallas as pl
from jax.experimental.pallas import tpu as pltpu

B, S, D = 2, 2048, 1024
DFF = 4 * D
N_EXPERTS = 8
VAR_EXPERTS = 4
FIXED_EXPERTS = N_EXPERTS - VAR_EXPERTS
TOP_K = 2
T = B * S
F_TOTAL = N_EXPERTS * DFF
LN_EPS = 1e-5

TM = 2048
TF = 512
N_I = T // TM
N_J = F_TOTAL // TF
TILES_PER_EXPERT = DFF // TF

TMA = 2048


def _router_body(rw_ref, x_ref, logits_ref, mask_ref):
    lg = jax.lax.dot_general(rw_ref[...], x_ref[...], (((1,), (1,)), ((), ())),
                             preferred_element_type=jnp.float32)
    logits_ref[...] = lg
    idx0 = jax.lax.broadcasted_iota(jnp.int32, (VAR_EXPERTS, lg.shape[1]), 0)
    rows = []
    for e in range(VAR_EXPERTS):
        le = lg[e:e + 1, :]
        beats = (lg > le) | ((lg == le) & (idx0 < e))
        rank = jnp.sum(beats.astype(jnp.float32), axis=0, keepdims=True)
        rows.append((rank < float(TOP_K)).astype(jnp.float32))
    ones = jnp.ones((FIXED_EXPERTS, lg.shape[1]), jnp.float32)
    mask_ref[...] = jnp.concatenate([ones] + rows, axis=0)


def _router_mask(xt, router_w):
    return pl.pallas_call(
        _router_body,
        grid=(T // TMA,),
        in_specs=[
            pl.BlockSpec((VAR_EXPERTS, D), lambda i: (0, 0)),
            pl.BlockSpec((TMA, D), lambda i: (i, 0)),
        ],
        out_specs=[
            pl.BlockSpec((VAR_EXPERTS, TMA), lambda i: (0, i)),
            pl.BlockSpec((N_EXPERTS, TMA), lambda i: (0, i)),
        ],
        out_shape=[
            jax.ShapeDtypeStruct((VAR_EXPERTS, T), jnp.float32),
            jax.ShapeDtypeStruct((N_EXPERTS, T), jnp.float32),
        ],
    )(router_w, xt)


def _moe_body(x_ref, w1_ref, b1_ref, w2_ref, mrow_ref, mall_ref, b2_ref,
              g_ref, bln_ref, o_ref):
    j = pl.program_id(1)
    nj = pl.num_programs(1)
    h = jax.lax.dot_general(x_ref[...], w1_ref[0], (((1,), (1,)), ((), ())),
                            preferred_element_type=jnp.float32)
    h = h + b1_ref[0]
    h = 0.5 * h * (1.0 + jax.lax.erf(h * 0.7071067811865476))
    gate = mrow_ref[0, 0, :]
    h = h * gate[:, None]
    contrib = jax.lax.dot_general(h, w2_ref[0], (((1,), (1,)), ((), ())),
                                  preferred_element_type=jnp.float32)

    @pl.when(j == 0)
    def _init():
        o_ref[...] = contrib

    @pl.when(j > 0)
    def _acc():
        o_ref[...] += contrib

    @pl.when(j == nj - 1)
    def _finalize():
        acc = o_ref[...]
        m_all = mall_ref[...].reshape(N_EXPERTS, acc.shape[0])
        bias = jax.lax.dot_general(m_all, b2_ref[...],
                                   (((0,), (0,)), ((), ())),
                                   preferred_element_type=jnp.float32)
        c = (acc + bias) * (1.0 / (FIXED_EXPERTS + TOP_K))
        mu = jnp.mean(c, axis=1, keepdims=True)
        var = jnp.mean((c - mu) ** 2, axis=1, keepdims=True)
        o_ref[...] = (c - mu) * jax.lax.rsqrt(var + LN_EPS) * g_ref[...] + bln_ref[...]


def _moe_mlp(xt, w1, b1, w2, b2, mask3, ln_g, ln_b):
    epj = TILES_PER_EXPERT

    def e_of(j):
        return j // epj

    def jj_of(j):
        return j % epj

    return pl.pallas_call(
        _moe_body,
        grid=(N_I, N_J),
        in_specs=[
            pl.BlockSpec((TM, D), lambda i, j: (i, 0)),
            pl.BlockSpec((1, TF, D), lambda i, j: (e_of(j), jj_of(j), 0)),
            pl.BlockSpec((1, 1, TF), lambda i, j: (j, 0, 0)),
            pl.BlockSpec((1, D, TF), lambda i, j: (e_of(j), 0, jj_of(j))),
            pl.BlockSpec((1, 1, TM), lambda i, j: (e_of(j), 0, i)),
            pl.BlockSpec((N_EXPERTS, 1, TM), lambda i, j: (0, 0, i)),
            pl.BlockSpec((N_EXPERTS, D), lambda i, j: (0, 0)),
            pl.BlockSpec((1, D), lambda i, j: (0, 0)),
            pl.BlockSpec((1, D), lambda i, j: (0, 0)),
        ],
        out_specs=pl.BlockSpec((TM, D), lambda i, j: (i, 0)),
        out_shape=jax.ShapeDtypeStruct((T, D), jnp.float32),
    )(xt, w1, b1.reshape(N_J, 1, TF), w2, mask3, mask3, b2, ln_g, ln_b)


@jax.jit
def kernel(x, router_w, w1, b1, w2, b2, ln_g, ln_b):
    xt = x.reshape(T, D)
    logitsT, maskT = _router_mask(xt, router_w)
    mask3 = maskT.reshape(N_EXPERTS, 1, T)
    out = _moe_mlp(xt, w1, b1, w2, b2, mask3,
                   ln_g.reshape(1, D), ln_b.reshape(1, D))
    router_logits = logitsT.T.reshape(B, S, VAR_EXPERTS)
    return (out.reshape(B, S, D), router_logits)
